# Initial kernel scaffold; baseline (speedup 1.0000x reference)
import jax
import jax.numpy as jnp
from jax.experimental import pallas as pl


def kernel(data_global, data_t0, data_t1, data_t2, data_t3, data_t4, data_t5, data_t6, data_t7, noise, mod_emb):
    raise NotImplementedError("write your pallas kernel here")


# speedup vs baseline: 4.3770x; 4.3770x over previous
import functools

import jax
import jax.numpy as jnp
import numpy as np
from jax.experimental import pallas as pl

B, T, D = 4, 2048, 768
NV = 8
NR = 4
NTOK = B * T
TB = 256
NBLK = NTOK // TB
PE_BLKS = T // TB


def _sinusoidal_pe(seq_len, d_model):
    pos = np.arange(seq_len, dtype=np.float32)[:, None]
    div = np.exp(np.arange(0, d_model, 2, dtype=np.float32) * (-np.log(10000.0) / d_model))
    pe = np.zeros((seq_len, d_model), dtype=np.float32)
    pe[:, 0::2] = np.sin(pos * div)
    pe[:, 1::2] = np.cos(pos * div)
    return pe


def _block_remain_kernel(g_ref, v0, v1, v2, v3, v4, v5, v6, v7,
                         noise_ref, emb_ref, pe_ref,
                         out_ref, masked_ref, revert_ref):
    valid = [v0, v1, v2, v3, v4, v5, v6, v7]
    n = noise_ref[...]
    j_iota = jax.lax.broadcasted_iota(jnp.int32, (1, NV), 1)

    ranks = jnp.zeros((TB, NV), jnp.int32)
    for i in range(NV):
        ni = n[:, i:i + 1]
        lt = (n < ni)
        eq = (n == ni) & (j_iota < i)
        rank_i = jnp.sum((lt | eq).astype(jnp.int32), axis=1, keepdims=True)
        ranks = ranks + rank_i * (j_iota == i).astype(jnp.int32)

    shuffle = jnp.zeros((TB, NV), jnp.int32)
    for i in range(NV):
        ri = ranks[:, i:i + 1]
        shuffle = shuffle + jnp.where(ri == j_iota, i, 0)

    masked_ref[...] = shuffle[:, NR:]
    revert_ref[...] = ranks

    pe = pe_ref[...]
    out_ref[:, 0, :] = g_ref[...] + emb_ref[0:1, :] + pe

    vp = [valid[i][...] + emb_ref[i + 1:i + 2, :] for i in range(NV)]
    for k in range(NR):
        sel = shuffle[:, k:k + 1]
        acc = vp[0]
        for i in range(1, NV):
            acc = jnp.where(sel == i, vp[i], acc)
        out_ref[:, k + 1, :] = acc + pe


@functools.partial(jax.jit, static_argnames=())
def _run(g, vs, noise, emb16, pe):
    tok_spec = pl.BlockSpec((TB, D), lambda i: (i, 0))
    out, masked, revert = pl.pallas_call(
        _block_remain_kernel,
        grid=(NBLK,),
        in_specs=[tok_spec] * (1 + NV) + [
            pl.BlockSpec((TB, NV), lambda i: (i, 0)),
            pl.BlockSpec((16, D), lambda i: (0, 0)),
            pl.BlockSpec((TB, D), lambda i: (i % PE_BLKS, 0)),
        ],
        out_specs=[
            pl.BlockSpec((TB, NR + 1, D), lambda i: (i, 0, 0)),
            pl.BlockSpec((TB, NV - NR), lambda i: (i, 0)),
            pl.BlockSpec((TB, NV), lambda i: (i, 0)),
        ],
        out_shape=[
            jax.ShapeDtypeStruct((NTOK, NR + 1, D), jnp.float32),
            jax.ShapeDtypeStruct((NTOK, NV - NR), jnp.int32),
            jax.ShapeDtypeStruct((NTOK, NV), jnp.int32),
        ],
    )(g, *vs, noise, emb16, pe)
    return out, masked, revert


def kernel(data_global, data_t0, data_t1, data_t2, data_t3, data_t4,
           data_t5, data_t6, data_t7, noise, mod_emb):
    g = data_global.reshape(NTOK, D)
    vs = [x.reshape(NTOK, D) for x in
          (data_t0, data_t1, data_t2, data_t3, data_t4, data_t5, data_t6, data_t7)]
    noise2 = noise.reshape(NTOK, NV)
    emb16 = jnp.zeros((16, D), jnp.float32).at[:NV + 1].set(mod_emb)
    pe = jnp.asarray(_sinusoidal_pe(T, D))
    out, masked, revert = _run(g, vs, noise2, emb16, pe)
    return (out.reshape(B, T, NR + 1, D),
            masked.reshape(B, T, NV - NR),
            revert.reshape(B, T, NV))
